# Initial kernel scaffold; baseline (speedup 1.0000x reference)
import jax
import jax.numpy as jnp
from jax.experimental import pallas as pl


def kernel(lang, embedding_table):
    raise NotImplementedError("write your pallas kernel here")


# speedup vs baseline: 7.9293x; 7.9293x over previous
import functools

import jax
import jax.numpy as jnp
from jax.experimental import pallas as pl
from jax.experimental.pallas import tpu as pltpu

_B, _L, _D = 16384, 200, 64
_BR = 64


def _tc_body(lang_ref, table_ref, out_ref):
    idx = lang_ref[...]
    t0 = table_ref[0, :]
    t1 = table_ref[1, :]
    sel = (idx[:, :, None] != 0)
    out_ref[...] = jnp.where(sel, t1[None, None, :], t0[None, None, :])


@jax.jit
def _tc_lookup(lang, table):
    grid = (_B // _BR,)
    return pl.pallas_call(
        _tc_body,
        grid=grid,
        in_specs=[
            pl.BlockSpec((_BR, _L), lambda i: (i, 0)),
            pl.BlockSpec((2, _D), lambda i: (0, 0)),
        ],
        out_specs=pl.BlockSpec((_BR, _L, _D), lambda i: (i, 0, 0)),
        out_shape=jax.ShapeDtypeStruct((_B, _L, _D), jnp.float32),
    )(lang, table)


def kernel(lang, embedding_table):
    return _tc_lookup(lang.astype(jnp.int32), embedding_table)
